# baseline (speedup 1.0000x reference)
import functools

import jax
import jax.numpy as jnp
from jax.experimental import pallas as pl
from jax.experimental.pallas import tpu as pltpu


def _route_weight_col(x, gw, gb, e):
    logits = jax.lax.dot_general(x, gw, (((1,), (1,)), ((), ())),
                                 preferred_element_type=jnp.float32)
    s = jax.nn.sigmoid(logits)
    sc = s + gb
    lane = jax.lax.broadcasted_iota(jnp.int32, sc.shape, 1)
    E = sc.shape[1]
    m1 = jnp.max(sc, axis=1, keepdims=True)
    eq1 = sc == m1
    i1 = jnp.min(jnp.where(eq1, lane, E), axis=1, keepdims=True)
    first1 = lane == i1
    scm = jnp.where(first1, -jnp.inf, sc)
    m2 = jnp.max(scm, axis=1, keepdims=True)
    eq2 = scm == m2
    i2 = jnp.min(jnp.where(eq2, lane, E), axis=1, keepdims=True)
    first2 = lane == i2
    sel = first1 | first2
    wsum = jnp.sum(jnp.where(sel, s, 0.0), axis=1, keepdims=True) + 1e-20
    return jnp.sum(jnp.where(sel & (lane == e), s, 0.0), axis=1,
                   keepdims=True) / wsum


def _moe_body(x_ref, xb_ref, gw_ref, gb_ref, wg_ref, wu_ref, wd_ref,
              wsg_ref, wsu_ref, wsd_ref, out_ref):
    e = pl.program_id(1)
    xb = xb_ref[...]
    w_e = _route_weight_col(x_ref[...], gw_ref[...], gb_ref[...], e)
    g = jax.lax.dot_general(xb, wg_ref[0], (((1,), (1,)), ((), ())),
                            preferred_element_type=jnp.float32)
    u = jax.lax.dot_general(xb, wu_ref[0], (((1,), (1,)), ((), ())),
                            preferred_element_type=jnp.float32)
    h = (g * jax.nn.sigmoid(g) * u).astype(jnp.bfloat16)
    y = jax.lax.dot_general(h, wd_ref[0], (((1,), (1,)), ((), ())),
                            preferred_element_type=jnp.float32)
    contrib = w_e * y

    @pl.when(e == 0)
    def _():
        gs = jax.lax.dot_general(xb, wsg_ref[...], (((1,), (1,)), ((), ())),
                                 preferred_element_type=jnp.float32)
        us = jax.lax.dot_general(xb, wsu_ref[...], (((1,), (1,)), ((), ())),
                                 preferred_element_type=jnp.float32)
        hs = (gs * jax.nn.sigmoid(gs) * us).astype(jnp.bfloat16)
        shared = jax.lax.dot_general(hs, wsd_ref[...], (((1,), (1,)), ((), ())),
                                     preferred_element_type=jnp.float32)
        out_ref[...] = shared + contrib

    @pl.when(e > 0)
    def _():
        out_ref[...] = out_ref[...] + contrib


def kernel(hidden_states, gate_w, gate_bias, w_gate, w_up, w_down,
           ws_gate, ws_up, ws_down):
    T, H = hidden_states.shape
    E, F, _ = w_gate.shape
    SF = ws_gate.shape[0]
    TBS = min(1024, T)
    gb2 = gate_bias.reshape(1, E)
    bf = jnp.bfloat16
    xb = hidden_states.astype(bf)
    grid = (T // TBS, E)
    return pl.pallas_call(
        _moe_body,
        grid=grid,
        in_specs=[
            pl.BlockSpec((TBS, H), lambda t, e: (t, 0)),
            pl.BlockSpec((TBS, H), lambda t, e: (t, 0)),
            pl.BlockSpec((E, H), lambda t, e: (0, 0)),
            pl.BlockSpec((1, E), lambda t, e: (0, 0)),
            pl.BlockSpec((1, F, H), lambda t, e: (e, 0, 0)),
            pl.BlockSpec((1, F, H), lambda t, e: (e, 0, 0)),
            pl.BlockSpec((1, H, F), lambda t, e: (e, 0, 0)),
            pl.BlockSpec((SF, H), lambda t, e: (0, 0)),
            pl.BlockSpec((SF, H), lambda t, e: (0, 0)),
            pl.BlockSpec((H, SF), lambda t, e: (0, 0)),
        ],
        out_specs=pl.BlockSpec((TBS, H), lambda t, e: (t, 0)),
        out_shape=jax.ShapeDtypeStruct((T, H), jnp.float32),
        compiler_params=pltpu.CompilerParams(
            dimension_semantics=("parallel", "arbitrary")),
    )(hidden_states, xb, gate_w, gb2,
      w_gate.astype(bf), w_up.astype(bf), w_down.astype(bf),
      ws_gate.astype(bf), ws_up.astype(bf), ws_down.astype(bf))


# speedup vs baseline: 1.3306x; 1.3306x over previous
import functools

import jax
import jax.numpy as jnp
from jax.experimental import pallas as pl
from jax.experimental.pallas import tpu as pltpu


def _route_weight_col(x, gw, gb, e):
    logits = jax.lax.dot_general(x, gw, (((1,), (1,)), ((), ())),
                                 preferred_element_type=jnp.float32)
    s = jax.nn.sigmoid(logits)
    sc = s + gb
    lane = jax.lax.broadcasted_iota(jnp.int32, sc.shape, 1)
    E = sc.shape[1]
    m1 = jnp.max(sc, axis=1, keepdims=True)
    eq1 = sc == m1
    i1 = jnp.min(jnp.where(eq1, lane, E), axis=1, keepdims=True)
    first1 = lane == i1
    scm = jnp.where(first1, -jnp.inf, sc)
    m2 = jnp.max(scm, axis=1, keepdims=True)
    eq2 = scm == m2
    i2 = jnp.min(jnp.where(eq2, lane, E), axis=1, keepdims=True)
    first2 = lane == i2
    sel = first1 | first2
    wsum = jnp.sum(jnp.where(sel, s, 0.0), axis=1, keepdims=True) + 1e-20
    return jnp.sum(jnp.where(sel & (lane == e), s, 0.0), axis=1,
                   keepdims=True) / wsum


def _moe_body(x_ref, gw_ref, gb_ref, wg_ref, wu_ref, wd_ref,
              wsg_ref, wsu_ref, wsd_ref, out_ref):
    e = pl.program_id(1)
    bf = jnp.bfloat16
    xb = x_ref[...].astype(bf)
    w_e = _route_weight_col(x_ref[...], gw_ref[...], gb_ref[...], e)
    g = jax.lax.dot_general(xb, wg_ref[0].astype(bf), (((1,), (1,)), ((), ())),
                            preferred_element_type=jnp.float32)
    u = jax.lax.dot_general(xb, wu_ref[0].astype(bf), (((1,), (1,)), ((), ())),
                            preferred_element_type=jnp.float32)
    h = (g * jax.nn.sigmoid(g) * u).astype(bf)
    y = jax.lax.dot_general(h, wd_ref[0].astype(bf), (((1,), (1,)), ((), ())),
                            preferred_element_type=jnp.float32)
    contrib = w_e * y

    @pl.when(e == 0)
    def _():
        gs = jax.lax.dot_general(xb, wsg_ref[...].astype(bf),
                                 (((1,), (1,)), ((), ())),
                                 preferred_element_type=jnp.float32)
        us = jax.lax.dot_general(xb, wsu_ref[...].astype(bf),
                                 (((1,), (1,)), ((), ())),
                                 preferred_element_type=jnp.float32)
        hs = (gs * jax.nn.sigmoid(gs) * us).astype(bf)
        shared = jax.lax.dot_general(hs, wsd_ref[...].astype(bf),
                                     (((1,), (1,)), ((), ())),
                                     preferred_element_type=jnp.float32)
        out_ref[...] = shared + contrib

    @pl.when(e > 0)
    def _():
        out_ref[...] = out_ref[...] + contrib


def kernel(hidden_states, gate_w, gate_bias, w_gate, w_up, w_down,
           ws_gate, ws_up, ws_down):
    T, H = hidden_states.shape
    E, F, _ = w_gate.shape
    SF = ws_gate.shape[0]
    TBS = min(1024, T)
    gb2 = gate_bias.reshape(1, E)
    grid = (T // TBS, E)
    return pl.pallas_call(
        _moe_body,
        grid=grid,
        in_specs=[
            pl.BlockSpec((TBS, H), lambda t, e: (t, 0)),
            pl.BlockSpec((E, H), lambda t, e: (0, 0)),
            pl.BlockSpec((1, E), lambda t, e: (0, 0)),
            pl.BlockSpec((1, F, H), lambda t, e: (e, 0, 0)),
            pl.BlockSpec((1, F, H), lambda t, e: (e, 0, 0)),
            pl.BlockSpec((1, H, F), lambda t, e: (e, 0, 0)),
            pl.BlockSpec((SF, H), lambda t, e: (0, 0)),
            pl.BlockSpec((SF, H), lambda t, e: (0, 0)),
            pl.BlockSpec((H, SF), lambda t, e: (0, 0)),
        ],
        out_specs=pl.BlockSpec((TBS, H), lambda t, e: (t, 0)),
        out_shape=jax.ShapeDtypeStruct((T, H), jnp.float32),
        compiler_params=pltpu.CompilerParams(
            dimension_semantics=("parallel", "arbitrary")),
    )(hidden_states, gate_w, gb2, w_gate, w_up, w_down,
      ws_gate, ws_up, ws_down)
